# baseline (speedup 1.0000x reference)
import functools

import numpy as np
import jax
import jax.numpy as jnp
from jax import lax
from jax.experimental import pallas as pl
from jax.experimental.pallas import tpu as pltpu
from jax.experimental.pallas import tpu_sc as plsc

B = 8
NPER = 32768
NQ = 64
K = 32
STEP = NPER // NQ
NTOT = B * NPER
NQTOT = B * NQ
NTILES = 32
QPT = NQTOT // NTILES
TPB = NTILES // B
NCHUNK = NPER // 16
INF = np.float32(3.4e38)


def _lex_lt(ka, ia, kb, ib):
    return (ka < kb) | ((ka == kb) & (ia < ib))


FCAP = 96


def _knn_body(pts, ctr, out, xs, ys, zs, cbuf, bufd, bufi, outv,
              fifoi, curs, thr):
    cid = lax.axis_index("c")
    sid = lax.axis_index("s")
    wid = sid * 2 + cid
    bi = wid // TPB
    qoff = (wid % TPB) * QPT
    base = bi * NPER

    pltpu.sync_copy(pts.at[pl.ds(base, NPER)], xs)
    pltpu.sync_copy(pts.at[pl.ds(NTOT + base, NPER)], ys)
    pltpu.sync_copy(pts.at[pl.ds(2 * NTOT + base, NPER)], zs)
    qbase = wid * QPT
    for c in range(3):
        pltpu.sync_copy(ctr.at[pl.ds(c * NQTOT + qbase, QPT)],
                        cbuf.at[pl.ds(c * QPT, QPT)])

    inf16 = jnp.full((16,), INF, jnp.float32)
    zero16 = jnp.zeros((16,), jnp.int32)
    for q in range(QPT):
        curs[q] = 0
        thr[pl.ds(q * 16, 16)] = inf16
        for h in range(2):
            bufd[pl.ds(q * K + h * 16, 16)] = inf16
            bufi[pl.ds(q * K + h * 16, 16)] = zero16

    cxv = cbuf[pl.ds(0 * QPT, 16)]
    cyv = cbuf[pl.ds(1 * QPT, 16)]
    czv = cbuf[pl.ds(2 * QPT, 16)]
    cxs = [cxv[q] for q in range(QPT)]
    cys = [cyv[q] for q in range(QPT)]
    czs = [czv[q] for q in range(QPT)]

    iota16 = lax.iota(jnp.int32, 16)

    def _merge(q, dm, ii):
        snew, inew = plsc.sort_key_val(dm, ii)
        b0d = bufd[pl.ds(q * K, 16)]
        b1d = bufd[pl.ds(q * K + 16, 16)]
        b0i = bufi[pl.ds(q * K, 16)]
        b1i = bufi[pl.ds(q * K + 16, 16)]
        rb1d = lax.rev(b1d, (0,))
        rb1i = lax.rev(b1i, (0,))
        lt = _lex_lt(snew, inew, rb1d, rb1i)
        ld = jnp.where(lt, snew, rb1d)
        li = jnp.where(lt, inew, rb1i)
        lsd, lsi = plsc.sort_key_val(ld, li)
        rld = lax.rev(lsd, (0,))
        rli = lax.rev(lsi, (0,))
        lt2 = _lex_lt(b0d, b0i, rld, rli)
        lod = jnp.where(lt2, b0d, rld)
        loi = jnp.where(lt2, b0i, rli)
        hid = jnp.where(lt2, rld, b0d)
        hii = jnp.where(lt2, rli, b0i)
        nb0d, nb0i = plsc.sort_key_val(lod, loi)
        nb1d, nb1i = plsc.sort_key_val(hid, hii)
        bufd[pl.ds(q * K, 16)] = nb0d
        bufd[pl.ds(q * K + 16, 16)] = nb1d
        bufi[pl.ds(q * K, 16)] = nb0i
        bufi[pl.ds(q * K + 16, 16)] = nb1i
        thr[pl.ds(q * 16, 16)] = jnp.full((16,), nb1d[15], jnp.float32)

    def _resq(q, ii):
        xg = plsc.load_gather(xs, [ii])
        yg = plsc.load_gather(ys, [ii])
        zg = plsc.load_gather(zs, [ii])
        dx = xg - cxs[q]
        dy = yg - cys[q]
        dz = zg - czs[q]
        return dx * dx + dy * dy + dz * dz

    def _drain(q, cur):
        qb = q * FCAP

        def _step(c):
            f0i = fifoi[pl.ds(qb, 16)]
            _merge(q, _resq(q, f0i), f0i)
            movei = [fifoi[pl.ds(qb + 16 * (j + 1), 16)] for j in range(5)]
            for j in range(5):
                fifoi[pl.ds(qb + 16 * j, 16)] = movei[j]
            return c - 16

        return lax.while_loop(lambda c: c >= 16, _step, cur)

    GQ = 8
    SUB = 4
    for g in range(QPT // GQ):
        qg = [g * GQ + i for i in range(GQ)]
        cxb = [jnp.full((16,), cxs[q], jnp.float32) for q in qg]
        cyb = [jnp.full((16,), cys[q], jnp.float32) for q in qg]
        czb = [jnp.full((16,), czs[q], jnp.float32) for q in qg]

        def _chunk(ci, carry, qg=qg, cxb=cxb, cyb=cyb, czb=czb):
            b64 = ci * (16 * SUB)
            pxs = [xs[pl.ds(b64 + 16 * s, 16)] for s in range(SUB)]
            pys = [ys[pl.ds(b64 + 16 * s, 16)] for s in range(SUB)]
            pzs = [zs[pl.ds(b64 + 16 * s, 16)] for s in range(SUB)]
            sqs = [[None] * SUB for _ in range(GQ)]
            masks = [[None] * SUB for _ in range(GQ)]
            for i, q in enumerate(qg):
                tv = thr[pl.ds(q * 16, 16)]
                for s in range(SUB):
                    dx = pxs[s] - cxb[i]
                    dy = pys[s] - cyb[i]
                    dz = pzs[s] - czb[i]
                    sq = dx * dx + dy * dy + dz * dz
                    sqs[i][s] = sq
                    masks[i][s] = sq < tv

            ivs = [b64 + 16 * s + iota16 for s in range(SUB)]
            ncurs = []
            for i, q in enumerate(qg):
                qb = q * FCAP
                cur = curs[q]
                for s in range(SUB):
                    m = masks[i][s]
                    cnt = plsc.all_reduce_population_count(m)[0]
                    plsc.store_compressed(
                        fifoi.at[pl.ds(qb + cur, 16)], ivs[s], mask=m)
                    cur = cur + cnt
                curs[q] = cur
                ncurs.append(cur)
            maxc = ncurs[0]
            for i in range(1, GQ):
                maxc = lax.max(maxc, ncurs[i])

            @pl.when(maxc >= 16)
            def _():
                for i, q in enumerate(qg):
                    curs[q] = _drain(q, ncurs[i])

            return carry

        lax.fori_loop(0, NCHUNK // SUB, _chunk, 0, unroll=2)

    for q in range(QPT):
        cur = curs[q]
        f0i = jnp.where(iota16 < cur, fifoi[pl.ds(q * FCAP, 16)], 0)
        dm = jnp.where(iota16 < cur, _resq(q, f0i), INF)
        _merge(q, dm, f0i)

    for q in range(QPT):
        for h in range(2):
            ii = bufi[pl.ds(q * K + h * 16, 16)]
            xg = plsc.load_gather(xs, [ii]) - cxs[q]
            yg = plsc.load_gather(ys, [ii]) - cys[q]
            zg = plsc.load_gather(zs, [ii]) - czs[q]
            outv[pl.ds(0 * QPT * K + q * K + h * 16, 16)] = xg
            outv[pl.ds(1 * QPT * K + q * K + h * 16, 16)] = yg
            outv[pl.ds(2 * QPT * K + q * K + h * 16, 16)] = zg
    obase = wid * QPT * K
    for c in range(3):
        pltpu.sync_copy(outv.at[pl.ds(c * QPT * K, QPT * K)],
                        out.at[pl.ds(c * NQTOT * K + obase, QPT * K)])


@functools.cache
def _knn_kernel():
    return pl.kernel(
        _knn_body,
        out_type=jax.ShapeDtypeStruct((3 * NQTOT * K,), jnp.float32),
        mesh=plsc.VectorSubcoreMesh(core_axis_name="c", subcore_axis_name="s"),
        compiler_params=pltpu.CompilerParams(needs_layout_passes=False),
        scratch_types=[
            pltpu.VMEM((NPER,), jnp.float32),
            pltpu.VMEM((NPER,), jnp.float32),
            pltpu.VMEM((NPER,), jnp.float32),
            pltpu.VMEM((3 * QPT,), jnp.float32),
            pltpu.VMEM((QPT * K,), jnp.float32),
            pltpu.VMEM((QPT * K,), jnp.int32),
            pltpu.VMEM((3 * QPT * K,), jnp.float32),
            pltpu.VMEM((QPT * FCAP,), jnp.int32),
            pltpu.SMEM((QPT,), jnp.int32),
            pltpu.VMEM((QPT * 16,), jnp.float32),
        ],
    )


def _knn(pts, ctr):
    return _knn_kernel()(pts, ctr)


def _gelu(x):
    return 0.5 * x * (1.0 + lax.erf(x * jnp.float32(0.7071067811865476)))


def _bn_batched(a, g, be):
    ar = a.reshape(B, NQ * K, a.shape[-1])
    mu = jnp.mean(ar, axis=1, keepdims=True)
    va = jnp.mean((ar - mu) * (ar - mu), axis=1, keepdims=True)
    ar = (ar - mu) / jnp.sqrt(va + 1e-5) * g + be
    return ar.reshape(a.shape)


def _mlp_body(rel, w1, b1, g1, be1, w2, b2, g2, be2, w3, b3, g3, be3, out):
    a = jnp.dot(rel[...], w1[...], preferred_element_type=jnp.float32) + b1[...]
    a = _gelu(_bn_batched(a, g1[...], be1[...]))
    a = jnp.dot(a, w2[...], preferred_element_type=jnp.float32) + b2[...]
    a = _gelu(_bn_batched(a, g2[...], be2[...]))
    a = jnp.dot(a, w3[...], preferred_element_type=jnp.float32) + b3[...]
    a = _bn_batched(a, g3[...], be3[...])
    out[...] = jnp.max(a.reshape(B, NQ, K, a.shape[-1]), axis=2)


def _mlp(rel, w1t, b1, g1, be1, w2t, b2, g2, be2, w3t, b3, g3, be3):
    return pl.pallas_call(
        _mlp_body,
        out_shape=jax.ShapeDtypeStruct((B, NQ, 384), jnp.float32),
    )(rel, w1t, b1.reshape(1, -1), g1.reshape(1, -1), be1.reshape(1, -1),
      w2t, b2.reshape(1, -1), g2.reshape(1, -1), be2.reshape(1, -1),
      w3t, b3.reshape(1, -1), g3.reshape(1, -1), be3.reshape(1, -1))


def kernel(xyz, features, batch, W1, b1, g1, be1, W2, b2, g2, be2,
           W3, b3, g3, be3):
    del features, batch
    pts = xyz.T.reshape(-1)
    centers = xyz.reshape(B, NPER, 3)[:, ::STEP, :]
    ctr = centers.reshape(NQTOT, 3).T.reshape(-1)
    relflat = _knn(pts, ctr)
    rel = relflat.reshape(3, NQTOT * K).T
    tokens = _mlp(rel, W1.T, b1, g1, be1, W2.T, b2, g2, be2, W3.T, b3, g3, be3)
    return tokens, centers


# speedup vs baseline: 2.1446x; 2.1446x over previous
import functools

import numpy as np
import jax
import jax.numpy as jnp
from jax import lax
from jax.experimental import pallas as pl
from jax.experimental.pallas import tpu as pltpu
from jax.experimental.pallas import tpu_sc as plsc

B = 8
NPER = 32768
NQ = 64
K = 32
STEP = NPER // NQ
NTOT = B * NPER
NQTOT = B * NQ
NTILES = 32
QPT = NQTOT // NTILES
TPB = NTILES // B
NCHUNK = NPER // 16
INF = np.float32(3.4e38)


def _lex_lt(ka, ia, kb, ib):
    return (ka < kb) | ((ka == kb) & (ia < ib))


FCAP = 96


def _knn_body(pts, ctr, out, xs, ys, zs, cbuf, bufd, bufi, outv,
              fifoi, curs, thr):
    cid = lax.axis_index("c")
    sid = lax.axis_index("s")
    wid = sid * 2 + cid
    bi = wid // TPB
    qoff = (wid % TPB) * QPT
    base = bi * NPER

    pltpu.sync_copy(pts.at[pl.ds(base, NPER)], xs)
    pltpu.sync_copy(pts.at[pl.ds(NTOT + base, NPER)], ys)
    pltpu.sync_copy(pts.at[pl.ds(2 * NTOT + base, NPER)], zs)
    qbase = wid * QPT
    for c in range(3):
        pltpu.sync_copy(ctr.at[pl.ds(c * NQTOT + qbase, QPT)],
                        cbuf.at[pl.ds(c * QPT, QPT)])

    inf16 = jnp.full((16,), INF, jnp.float32)
    zero16 = jnp.zeros((16,), jnp.int32)
    for q in range(QPT):
        curs[q] = 0
        thr[pl.ds(q * 16, 16)] = inf16
        for h in range(2):
            bufd[pl.ds(q * K + h * 16, 16)] = inf16
            bufi[pl.ds(q * K + h * 16, 16)] = zero16

    cxv = cbuf[pl.ds(0 * QPT, 16)]
    cyv = cbuf[pl.ds(1 * QPT, 16)]
    czv = cbuf[pl.ds(2 * QPT, 16)]
    cxs = [cxv[q] for q in range(QPT)]
    cys = [cyv[q] for q in range(QPT)]
    czs = [czv[q] for q in range(QPT)]

    iota16 = lax.iota(jnp.int32, 16)

    def _merge(q, dm, ii):
        snew, inew = plsc.sort_key_val(dm, ii)
        b0d = bufd[pl.ds(q * K, 16)]
        b1d = bufd[pl.ds(q * K + 16, 16)]
        b0i = bufi[pl.ds(q * K, 16)]
        b1i = bufi[pl.ds(q * K + 16, 16)]
        rb1d = lax.rev(b1d, (0,))
        rb1i = lax.rev(b1i, (0,))
        lt = _lex_lt(snew, inew, rb1d, rb1i)
        ld = jnp.where(lt, snew, rb1d)
        li = jnp.where(lt, inew, rb1i)
        lsd, lsi = plsc.sort_key_val(ld, li)
        rld = lax.rev(lsd, (0,))
        rli = lax.rev(lsi, (0,))
        lt2 = _lex_lt(b0d, b0i, rld, rli)
        lod = jnp.where(lt2, b0d, rld)
        loi = jnp.where(lt2, b0i, rli)
        hid = jnp.where(lt2, rld, b0d)
        hii = jnp.where(lt2, rli, b0i)
        nb0d, nb0i = plsc.sort_key_val(lod, loi)
        nb1d, nb1i = plsc.sort_key_val(hid, hii)
        bufd[pl.ds(q * K, 16)] = nb0d
        bufd[pl.ds(q * K + 16, 16)] = nb1d
        bufi[pl.ds(q * K, 16)] = nb0i
        bufi[pl.ds(q * K + 16, 16)] = nb1i
        thr[pl.ds(q * 16, 16)] = jnp.full((16,), nb1d[15], jnp.float32)

    def _resq(q, ii):
        xg = plsc.load_gather(xs, [ii])
        yg = plsc.load_gather(ys, [ii])
        zg = plsc.load_gather(zs, [ii])
        dx = xg - cxs[q]
        dy = yg - cys[q]
        dz = zg - czs[q]
        return dx * dx + dy * dy + dz * dz

    def _drain(q, cur):
        qb = q * FCAP

        def _step(c):
            f0i = fifoi[pl.ds(qb, 16)]
            _merge(q, _resq(q, f0i), f0i)
            movei = [fifoi[pl.ds(qb + 16 * (j + 1), 16)] for j in range(5)]
            for j in range(5):
                fifoi[pl.ds(qb + 16 * j, 16)] = movei[j]
            return c - 16

        return lax.while_loop(lambda c: c >= 16, _step, cur)

    GQ = 8
    SUB = 4
    for g in range(QPT // GQ):
        qg = [g * GQ + i for i in range(GQ)]
        cxb = [jnp.full((16,), cxs[q], jnp.float32) for q in qg]
        cyb = [jnp.full((16,), cys[q], jnp.float32) for q in qg]
        czb = [jnp.full((16,), czs[q], jnp.float32) for q in qg]

        def _chunk(ci, carry, qg=qg, cxb=cxb, cyb=cyb, czb=czb):
            b64 = ci * (16 * SUB)
            pxs = [xs[pl.ds(b64 + 16 * s, 16)] for s in range(SUB)]
            pys = [ys[pl.ds(b64 + 16 * s, 16)] for s in range(SUB)]
            pzs = [zs[pl.ds(b64 + 16 * s, 16)] for s in range(SUB)]
            sqs = [[None] * SUB for _ in range(GQ)]
            masks = [[None] * SUB for _ in range(GQ)]
            for i, q in enumerate(qg):
                tv = thr[pl.ds(q * 16, 16)]
                for s in range(SUB):
                    dx = pxs[s] - cxb[i]
                    dy = pys[s] - cyb[i]
                    dz = pzs[s] - czb[i]
                    sq = dx * dx + dy * dy + dz * dz
                    sqs[i][s] = sq
                    masks[i][s] = sq < tv

            ivs = [b64 + 16 * s + iota16 for s in range(SUB)]
            ncurs = []
            for i, q in enumerate(qg):
                qb = q * FCAP
                cur = curs[q]
                for s in range(SUB):
                    m = masks[i][s]
                    cnt = plsc.all_reduce_population_count(m)[0]
                    plsc.store_compressed(
                        fifoi.at[pl.ds(qb + cur, 16)], ivs[s], mask=m)
                    cur = cur + cnt
                curs[q] = cur
                ncurs.append(cur)
            maxc = ncurs[0]
            for i in range(1, GQ):
                maxc = lax.max(maxc, ncurs[i])

            @pl.when(maxc >= 16)
            def _():
                for i, q in enumerate(qg):
                    curs[q] = _drain(q, ncurs[i])

            return carry

        lax.fori_loop(0, NCHUNK // SUB, _chunk, 0)

    for q in range(QPT):
        cur = curs[q]
        f0i = jnp.where(iota16 < cur, fifoi[pl.ds(q * FCAP, 16)], 0)
        dm = jnp.where(iota16 < cur, _resq(q, f0i), INF)
        _merge(q, dm, f0i)

    for q in range(QPT):
        for h in range(2):
            ii = bufi[pl.ds(q * K + h * 16, 16)]
            xg = plsc.load_gather(xs, [ii]) - cxs[q]
            yg = plsc.load_gather(ys, [ii]) - cys[q]
            zg = plsc.load_gather(zs, [ii]) - czs[q]
            outv[pl.ds(0 * QPT * K + q * K + h * 16, 16)] = xg
            outv[pl.ds(1 * QPT * K + q * K + h * 16, 16)] = yg
            outv[pl.ds(2 * QPT * K + q * K + h * 16, 16)] = zg
    obase = wid * QPT * K
    for c in range(3):
        pltpu.sync_copy(outv.at[pl.ds(c * QPT * K, QPT * K)],
                        out.at[pl.ds(c * NQTOT * K + obase, QPT * K)])


@functools.cache
def _knn_kernel():
    return pl.kernel(
        _knn_body,
        out_type=jax.ShapeDtypeStruct((3 * NQTOT * K,), jnp.float32),
        mesh=plsc.VectorSubcoreMesh(core_axis_name="c", subcore_axis_name="s"),
        compiler_params=pltpu.CompilerParams(needs_layout_passes=False),
        scratch_types=[
            pltpu.VMEM((NPER,), jnp.float32),
            pltpu.VMEM((NPER,), jnp.float32),
            pltpu.VMEM((NPER,), jnp.float32),
            pltpu.VMEM((3 * QPT,), jnp.float32),
            pltpu.VMEM((QPT * K,), jnp.float32),
            pltpu.VMEM((QPT * K,), jnp.int32),
            pltpu.VMEM((3 * QPT * K,), jnp.float32),
            pltpu.VMEM((QPT * FCAP,), jnp.int32),
            pltpu.SMEM((QPT,), jnp.int32),
            pltpu.VMEM((QPT * 16,), jnp.float32),
        ],
    )


def _knn(pts, ctr):
    return _knn_kernel()(pts, ctr)


def _gelu(x):
    return 0.5 * x * (1.0 + lax.erf(x * jnp.float32(0.7071067811865476)))


def _bn_batched(a, g, be):
    ar = a.reshape(B, NQ * K, a.shape[-1])
    mu = jnp.mean(ar, axis=1, keepdims=True)
    va = jnp.mean((ar - mu) * (ar - mu), axis=1, keepdims=True)
    ar = (ar - mu) / jnp.sqrt(va + 1e-5) * g + be
    return ar.reshape(a.shape)


def _mlp_body(rel, w1, b1, g1, be1, w2, b2, g2, be2, w3, b3, g3, be3, out):
    a = jnp.dot(rel[...], w1[...], preferred_element_type=jnp.float32) + b1[...]
    a = _gelu(_bn_batched(a, g1[...], be1[...]))
    a = jnp.dot(a, w2[...], preferred_element_type=jnp.float32) + b2[...]
    a = _gelu(_bn_batched(a, g2[...], be2[...]))
    a = jnp.dot(a, w3[...], preferred_element_type=jnp.float32) + b3[...]
    a = _bn_batched(a, g3[...], be3[...])
    out[...] = jnp.max(a.reshape(B, NQ, K, a.shape[-1]), axis=2)


def _mlp(rel, w1t, b1, g1, be1, w2t, b2, g2, be2, w3t, b3, g3, be3):
    return pl.pallas_call(
        _mlp_body,
        out_shape=jax.ShapeDtypeStruct((B, NQ, 384), jnp.float32),
    )(rel, w1t, b1.reshape(1, -1), g1.reshape(1, -1), be1.reshape(1, -1),
      w2t, b2.reshape(1, -1), g2.reshape(1, -1), be2.reshape(1, -1),
      w3t, b3.reshape(1, -1), g3.reshape(1, -1), be3.reshape(1, -1))


def kernel(xyz, features, batch, W1, b1, g1, be1, W2, b2, g2, be2,
           W3, b3, g3, be3):
    del features, batch
    pts = xyz.T.reshape(-1)
    centers = xyz.reshape(B, NPER, 3)[:, ::STEP, :]
    ctr = centers.reshape(NQTOT, 3).T.reshape(-1)
    relflat = _knn(pts, ctr)
    rel = relflat.reshape(3, NQTOT * K).T
    tokens = _mlp(rel, W1.T, b1, g1, be1, W2.T, b2, g2, be2, W3.T, b3, g3, be3)
    return tokens, centers


# speedup vs baseline: 2.3641x; 1.1024x over previous
import functools

import numpy as np
import jax
import jax.numpy as jnp
from jax import lax
from jax.experimental import pallas as pl
from jax.experimental.pallas import tpu as pltpu
from jax.experimental.pallas import tpu_sc as plsc

B = 8
NPER = 32768
NQ = 64
K = 32
STEP = NPER // NQ
NTOT = B * NPER
NQTOT = B * NQ
NTILES = 32
QPT = NQTOT // NTILES
TPB = NTILES // B
NCHUNK = NPER // 16
INF = np.float32(3.4e38)


def _lex_lt(ka, ia, kb, ib):
    return (ka < kb) | ((ka == kb) & (ia < ib))


FCAP = 96


def _knn_body(pts, ctr, out, xs, ys, zs, cbuf, bufd, bufi, outv,
              fifoi, curs, thr):
    cid = lax.axis_index("c")
    sid = lax.axis_index("s")
    wid = sid * 2 + cid
    bi = wid // TPB
    qoff = (wid % TPB) * QPT
    base = bi * NPER

    pltpu.sync_copy(pts.at[pl.ds(base, NPER)], xs)
    pltpu.sync_copy(pts.at[pl.ds(NTOT + base, NPER)], ys)
    pltpu.sync_copy(pts.at[pl.ds(2 * NTOT + base, NPER)], zs)
    qbase = wid * QPT
    for c in range(3):
        pltpu.sync_copy(ctr.at[pl.ds(c * NQTOT + qbase, QPT)],
                        cbuf.at[pl.ds(c * QPT, QPT)])

    inf16 = jnp.full((16,), INF, jnp.float32)
    zero16 = jnp.zeros((16,), jnp.int32)
    for q in range(QPT):
        curs[q] = 0
        thr[pl.ds(q * 16, 16)] = inf16
        for h in range(2):
            bufd[pl.ds(q * K + h * 16, 16)] = inf16
            bufi[pl.ds(q * K + h * 16, 16)] = zero16

    cxv = cbuf[pl.ds(0 * QPT, 16)]
    cyv = cbuf[pl.ds(1 * QPT, 16)]
    czv = cbuf[pl.ds(2 * QPT, 16)]
    cxs = [cxv[q] for q in range(QPT)]
    cys = [cyv[q] for q in range(QPT)]
    czs = [czv[q] for q in range(QPT)]

    iota16 = lax.iota(jnp.int32, 16)

    def _merge(q, dm, ii):
        snew, inew = plsc.sort_key_val(dm, ii)
        b0d = bufd[pl.ds(q * K, 16)]
        b1d = bufd[pl.ds(q * K + 16, 16)]
        b0i = bufi[pl.ds(q * K, 16)]
        b1i = bufi[pl.ds(q * K + 16, 16)]
        rb1d = lax.rev(b1d, (0,))
        rb1i = lax.rev(b1i, (0,))
        lt = _lex_lt(snew, inew, rb1d, rb1i)
        ld = jnp.where(lt, snew, rb1d)
        li = jnp.where(lt, inew, rb1i)
        lsd, lsi = plsc.sort_key_val(ld, li)
        rld = lax.rev(lsd, (0,))
        rli = lax.rev(lsi, (0,))
        lt2 = _lex_lt(b0d, b0i, rld, rli)
        lod = jnp.where(lt2, b0d, rld)
        loi = jnp.where(lt2, b0i, rli)
        hid = jnp.where(lt2, rld, b0d)
        hii = jnp.where(lt2, rli, b0i)
        nb0d, nb0i = plsc.sort_key_val(lod, loi)
        nb1d, nb1i = plsc.sort_key_val(hid, hii)
        bufd[pl.ds(q * K, 16)] = nb0d
        bufd[pl.ds(q * K + 16, 16)] = nb1d
        bufi[pl.ds(q * K, 16)] = nb0i
        bufi[pl.ds(q * K + 16, 16)] = nb1i
        thr[pl.ds(q * 16, 16)] = jnp.full((16,), nb1d[15], jnp.float32)

    def _resq(q, ii):
        xg = plsc.load_gather(xs, [ii])
        yg = plsc.load_gather(ys, [ii])
        zg = plsc.load_gather(zs, [ii])
        dx = xg - cxs[q]
        dy = yg - cys[q]
        dz = zg - czs[q]
        return dx * dx + dy * dy + dz * dz

    def _drain(q, cur):
        qb = q * FCAP

        def _step(c):
            f0i = fifoi[pl.ds(qb, 16)]
            _merge(q, _resq(q, f0i), f0i)
            movei = [fifoi[pl.ds(qb + 16 * (j + 1), 16)] for j in range(5)]
            for j in range(5):
                fifoi[pl.ds(qb + 16 * j, 16)] = movei[j]
            return c - 16

        return lax.while_loop(lambda c: c >= 16, _step, cur)

    GQ = 4
    SUB = 4
    for g in range(QPT // GQ):
        qg = [g * GQ + i for i in range(GQ)]
        cxb = [jnp.full((16,), cxs[q], jnp.float32) for q in qg]
        cyb = [jnp.full((16,), cys[q], jnp.float32) for q in qg]
        czb = [jnp.full((16,), czs[q], jnp.float32) for q in qg]

        def _chunk(ci, carry, qg=qg, cxb=cxb, cyb=cyb, czb=czb):
            b64 = ci * (16 * SUB)
            pxs = [xs[pl.ds(b64 + 16 * s, 16)] for s in range(SUB)]
            pys = [ys[pl.ds(b64 + 16 * s, 16)] for s in range(SUB)]
            pzs = [zs[pl.ds(b64 + 16 * s, 16)] for s in range(SUB)]
            sqs = [[None] * SUB for _ in range(GQ)]
            masks = [[None] * SUB for _ in range(GQ)]
            for i, q in enumerate(qg):
                tv = thr[pl.ds(q * 16, 16)]
                for s in range(SUB):
                    dx = pxs[s] - cxb[i]
                    dy = pys[s] - cyb[i]
                    dz = pzs[s] - czb[i]
                    sq = dx * dx + dy * dy + dz * dz
                    sqs[i][s] = sq
                    masks[i][s] = sq < tv

            ivs = [b64 + 16 * s + iota16 for s in range(SUB)]
            ncurs = []
            for i, q in enumerate(qg):
                qb = q * FCAP
                cur = curs[q]
                for s in range(SUB):
                    m = masks[i][s]
                    cnt = plsc.all_reduce_population_count(m)[0]
                    plsc.store_compressed(
                        fifoi.at[pl.ds(qb + cur, 16)], ivs[s], mask=m)
                    cur = cur + cnt
                curs[q] = cur
                ncurs.append(cur)
            maxc = ncurs[0]
            for i in range(1, GQ):
                maxc = lax.max(maxc, ncurs[i])

            @pl.when(maxc >= 16)
            def _():
                for i, q in enumerate(qg):
                    curs[q] = _drain(q, ncurs[i])

            return carry

        lax.fori_loop(0, NCHUNK // SUB, _chunk, 0)

    for q in range(QPT):
        cur = curs[q]
        f0i = jnp.where(iota16 < cur, fifoi[pl.ds(q * FCAP, 16)], 0)
        dm = jnp.where(iota16 < cur, _resq(q, f0i), INF)
        _merge(q, dm, f0i)

    for q in range(QPT):
        for h in range(2):
            ii = bufi[pl.ds(q * K + h * 16, 16)]
            xg = plsc.load_gather(xs, [ii]) - cxs[q]
            yg = plsc.load_gather(ys, [ii]) - cys[q]
            zg = plsc.load_gather(zs, [ii]) - czs[q]
            outv[pl.ds(0 * QPT * K + q * K + h * 16, 16)] = xg
            outv[pl.ds(1 * QPT * K + q * K + h * 16, 16)] = yg
            outv[pl.ds(2 * QPT * K + q * K + h * 16, 16)] = zg
    obase = wid * QPT * K
    for c in range(3):
        pltpu.sync_copy(outv.at[pl.ds(c * QPT * K, QPT * K)],
                        out.at[pl.ds(c * NQTOT * K + obase, QPT * K)])


@functools.cache
def _knn_kernel():
    return pl.kernel(
        _knn_body,
        out_type=jax.ShapeDtypeStruct((3 * NQTOT * K,), jnp.float32),
        mesh=plsc.VectorSubcoreMesh(core_axis_name="c", subcore_axis_name="s"),
        compiler_params=pltpu.CompilerParams(needs_layout_passes=False),
        scratch_types=[
            pltpu.VMEM((NPER,), jnp.float32),
            pltpu.VMEM((NPER,), jnp.float32),
            pltpu.VMEM((NPER,), jnp.float32),
            pltpu.VMEM((3 * QPT,), jnp.float32),
            pltpu.VMEM((QPT * K,), jnp.float32),
            pltpu.VMEM((QPT * K,), jnp.int32),
            pltpu.VMEM((3 * QPT * K,), jnp.float32),
            pltpu.VMEM((QPT * FCAP,), jnp.int32),
            pltpu.SMEM((QPT,), jnp.int32),
            pltpu.VMEM((QPT * 16,), jnp.float32),
        ],
    )


def _knn(pts, ctr):
    return _knn_kernel()(pts, ctr)


def _gelu(x):
    return 0.5 * x * (1.0 + lax.erf(x * jnp.float32(0.7071067811865476)))


def _bn_batched(a, g, be):
    ar = a.reshape(B, NQ * K, a.shape[-1])
    mu = jnp.mean(ar, axis=1, keepdims=True)
    va = jnp.mean((ar - mu) * (ar - mu), axis=1, keepdims=True)
    ar = (ar - mu) / jnp.sqrt(va + 1e-5) * g + be
    return ar.reshape(a.shape)


def _mlp_body(rel, w1, b1, g1, be1, w2, b2, g2, be2, w3, b3, g3, be3, out):
    a = jnp.dot(rel[...], w1[...], preferred_element_type=jnp.float32) + b1[...]
    a = _gelu(_bn_batched(a, g1[...], be1[...]))
    a = jnp.dot(a, w2[...], preferred_element_type=jnp.float32) + b2[...]
    a = _gelu(_bn_batched(a, g2[...], be2[...]))
    a = jnp.dot(a, w3[...], preferred_element_type=jnp.float32) + b3[...]
    a = _bn_batched(a, g3[...], be3[...])
    out[...] = jnp.max(a.reshape(B, NQ, K, a.shape[-1]), axis=2)


def _mlp(rel, w1t, b1, g1, be1, w2t, b2, g2, be2, w3t, b3, g3, be3):
    return pl.pallas_call(
        _mlp_body,
        out_shape=jax.ShapeDtypeStruct((B, NQ, 384), jnp.float32),
    )(rel, w1t, b1.reshape(1, -1), g1.reshape(1, -1), be1.reshape(1, -1),
      w2t, b2.reshape(1, -1), g2.reshape(1, -1), be2.reshape(1, -1),
      w3t, b3.reshape(1, -1), g3.reshape(1, -1), be3.reshape(1, -1))


def kernel(xyz, features, batch, W1, b1, g1, be1, W2, b2, g2, be2,
           W3, b3, g3, be3):
    del features, batch
    pts = xyz.T.reshape(-1)
    centers = xyz.reshape(B, NPER, 3)[:, ::STEP, :]
    ctr = centers.reshape(NQTOT, 3).T.reshape(-1)
    relflat = _knn(pts, ctr)
    rel = relflat.reshape(3, NQTOT * K).T
    tokens = _mlp(rel, W1.T, b1, g1, be1, W2.T, b2, g2, be2, W3.T, b3, g3, be3)
    return tokens, centers
